# baseline (speedup 1.0000x reference)
import functools

import jax
import jax.numpy as jnp
from jax import lax
from jax.experimental import pallas as pl
from jax.experimental.pallas import tpu as pltpu
from jax.experimental.pallas import tpu_sc as plsc

QN = 16
DN = 16
KN = 1_000_000
NNBR = 5
NCLS = 32

BLK = 40000
RB = BLK // 8
GRID = KN // BLK

NEGF = -3.0e38
BIGI = 2**31 - 1


def _topk_extract(cv, ci, n):
    vs, is_ = [], []
    for _ in range(n):
        m = jnp.max(cv, axis=0, keepdims=True)
        sel = jnp.where(cv == m, ci, BIGI)
        ii = jnp.min(sel, axis=0, keepdims=True)
        vs.append(m)
        is_.append(ii)
        cv = jnp.where(ci == ii, NEGF, cv)
    return jnp.concatenate(vs, axis=0), jnp.concatenate(is_, axis=0)


def _tc_body(q_ref, qbd_ref, obd_ref, x_ref, outv_ref, outi_ref,
             sv_ref, si_ref, pat_ref):
    i = pl.program_id(0)

    @pl.when(i == 0)
    def _init():
        sv_ref[...] = jnp.full((8, 128), NEGF, jnp.float32)
        si_ref[...] = jnp.full((8, 128), BIGI, jnp.int32)
        row = lax.broadcasted_iota(jnp.int32, (RB, 128), 0)
        lane = lax.broadcasted_iota(jnp.int32, (RB, 128), 1)
        pat_ref[...] = row * 8 + lane // 16

    x = x_ref[...]
    dots = lax.dot_general(x.astype(jnp.bfloat16),
                           qbd_ref[...].astype(jnp.bfloat16),
                           (((1,), (0,)), ((), ())),
                           preferred_element_type=jnp.float32)
    xn = lax.dot_general(x * x, obd_ref[...], (((1,), (0,)), ((), ())),
                         preferred_element_type=jnp.float32,
                         precision=lax.Precision.HIGHEST)
    cv = dots - xn
    ci = pat_ref[...] + i * BLK

    bv, bi = _topk_extract(cv, ci, NNBR)
    mv = jnp.concatenate([sv_ref[0:NNBR, :], bv], axis=0)
    mi = jnp.concatenate([si_ref[0:NNBR, :], bi], axis=0)
    nv, ni = _topk_extract(mv, mi, NNBR)
    sv_ref[0:NNBR, :] = nv
    si_ref[0:NNBR, :] = ni

    @pl.when(i == GRID - 1)
    def _fin():
        fv = sv_ref[0:NNBR, :]
        fi = si_ref[0:NNBR, :]
        cvf = jnp.concatenate([fv[:, g * 16:(g + 1) * 16] for g in range(8)],
                              axis=0)
        cif = jnp.concatenate([fi[:, g * 16:(g + 1) * 16] for g in range(8)],
                              axis=0)
        qq = q_ref[...]
        qn = lax.dot_general(jnp.ones((1, QN), jnp.float32), qq * qq,
                             (((1,), (1,)), ((), ())),
                             preferred_element_type=jnp.float32,
                             precision=lax.Precision.HIGHEST)
        tv, ti = _topk_extract(cvf, cif, NNBR)
        outv_ref[0:NNBR, :] = tv - qn
        outi_ref[0:NNBR, :] = ti


def _tc_topk(q, qbd, obd, xr):
    return pl.pallas_call(
        _tc_body,
        grid=(GRID,),
        in_specs=[
            pl.BlockSpec((QN, DN), lambda i: (0, 0)),
            pl.BlockSpec((128, 128), lambda i: (0, 0)),
            pl.BlockSpec((128, 128), lambda i: (0, 0)),
            pl.BlockSpec((RB, 128), lambda i: (i, 0)),
        ],
        out_specs=[
            pl.BlockSpec((8, QN), lambda i: (0, 0)),
            pl.BlockSpec((8, QN), lambda i: (0, 0)),
        ],
        out_shape=[
            jax.ShapeDtypeStruct((8, QN), jnp.float32),
            jax.ShapeDtypeStruct((8, QN), jnp.int32),
        ],
        scratch_shapes=[
            pltpu.VMEM((8, 128), jnp.float32),
            pltpu.VMEM((8, 128), jnp.int32),
            pltpu.VMEM((RB, 128), jnp.int32),
        ],
    )(q, qbd, obd, xr)


def _sc_vote_body(labels_hbm, idx_hbm, pred_hbm, idx_v, lab_v, pred_v, sem):
    c = lax.axis_index("c")
    s = lax.axis_index("s")
    wid = s * 2 + c

    @pl.when(wid == 0)
    def _():
        pltpu.sync_copy(idx_hbm, idx_v)
        pltpu.async_copy(labels_hbm.at[idx_v], lab_v, sem).wait()
        labs = [lab_v[pl.ds(j * QN, QN)] for j in range(NNBR)]
        best = jnp.full((QN,), -1, jnp.int32)
        pred = jnp.full((QN,), 0, jnp.int32)
        one = jnp.full((QN,), 1, jnp.int32)
        zero = jnp.full((QN,), 0, jnp.int32)
        for cc in range(NCLS):
            cc_v = jnp.full((QN,), cc, jnp.int32)
            cnt = zero
            for j in range(NNBR):
                cnt = cnt + jnp.where(labs[j] == cc_v, one, zero)
            better = cnt > best
            best = jnp.where(better, cnt, best)
            pred = jnp.where(better, cc_v, pred)
        pred_v[...] = pred
        pltpu.sync_copy(pred_v, pred_hbm)


@functools.cache
def _sc_vote():
    return pl.kernel(
        _sc_vote_body,
        out_type=jax.ShapeDtypeStruct((QN,), jnp.int32),
        mesh=plsc.VectorSubcoreMesh(core_axis_name="c", subcore_axis_name="s"),
        scratch_types=[
            pltpu.VMEM((NNBR * QN,), jnp.int32),
            pltpu.VMEM((NNBR * QN,), jnp.int32),
            pltpu.VMEM((QN,), jnp.int32),
            pltpu.SemaphoreType.DMA,
        ],
    )


def kernel(test_query_embedding, train_x, train_labels):
    q = test_query_embedding
    eye8 = jnp.eye(8, dtype=jnp.float32)
    qbd = jnp.kron(eye8, 2.0 * q.T)
    obd = jnp.kron(eye8, jnp.ones((DN, QN), jnp.float32))
    xr = train_x.reshape(KN // 8, 128)

    outv, outi = _tc_topk(q, qbd, obd, xr)
    neg_topk_dist = outv[:NNBR].T
    idx_flat = outi[:NNBR].reshape(-1)

    pred = _sc_vote()(train_labels, idx_flat)
    return pred, neg_topk_dist


# speedup vs baseline: 3.2297x; 3.2297x over previous
import functools

import jax
import jax.numpy as jnp
from jax import lax
from jax.experimental import pallas as pl
from jax.experimental.pallas import tpu as pltpu
from jax.experimental.pallas import tpu_sc as plsc

QN = 16
DN = 16
KN = 1_000_000
NNBR = 5
NCLS = 32

BLK = 16384
NSPL = 4
QW = BLK // NSPL
GRID = (KN + BLK - 1) // BLK

NEGF = -3.0e38
BIGI = 2**31 - 1


def _topk_extract2(cv, ci, n):
    vs, is_ = [], []
    for _ in range(n):
        m = jnp.max(cv, axis=1, keepdims=True)
        sel = jnp.where(cv == m, ci, BIGI)
        ii = jnp.min(sel, axis=1, keepdims=True)
        vs.append(m)
        is_.append(ii)
        cv = jnp.where(ci == ii, NEGF, cv)
    return jnp.concatenate(vs, axis=1), jnp.concatenate(is_, axis=1)


def _tc_body(q_ref, xt_ref, outv_ref, outi_ref, sv_ref, si_ref):
    i = pl.program_id(0)

    @pl.when(i == 0)
    def _init():
        sv_ref[...] = jnp.full((QN, 8), NEGF, jnp.float32)
        si_ref[...] = jnp.full((QN, 8), BIGI, jnp.int32)

    xt = xt_ref[...]
    q2bf = (q_ref[...] * 2.0).astype(jnp.bfloat16)
    dots = lax.dot_general(q2bf, xt.astype(jnp.bfloat16),
                           (((1,), (0,)), ((), ())),
                           preferred_element_type=jnp.float32)
    xn = jnp.sum(xt * xt, axis=0, keepdims=True)
    neg = dots - xn

    bvs, bis = [sv_ref[:, 0:NNBR]], [si_ref[:, 0:NNBR]]
    for qd in range(NSPL):
        cv = neg[:, qd * QW:(qd + 1) * QW]
        ci = (lax.broadcasted_iota(jnp.int32, (QN, QW), 1)
              + (i * BLK + qd * QW))
        cv = jnp.where(ci < KN, cv, NEGF)
        bv, bi = _topk_extract2(cv, ci, NNBR)
        bvs.append(bv)
        bis.append(bi)
    mv = jnp.concatenate(bvs, axis=1)
    mi = jnp.concatenate(bis, axis=1)
    nv, ni = _topk_extract2(mv, mi, NNBR)
    sv_ref[:, 0:NNBR] = nv
    si_ref[:, 0:NNBR] = ni

    @pl.when(i == GRID - 1)
    def _fin():
        qq = q_ref[...]
        qn = jnp.sum(qq * qq, axis=1, keepdims=True)
        outv_ref[:, 0:NNBR] = sv_ref[:, 0:NNBR] - qn
        outi_ref[:, 0:NNBR] = si_ref[:, 0:NNBR]


def _tc_topk(q, xt):
    return pl.pallas_call(
        _tc_body,
        grid=(GRID,),
        in_specs=[
            pl.BlockSpec((QN, DN), lambda i: (0, 0)),
            pl.BlockSpec((QN, BLK), lambda i: (0, i)),
        ],
        out_specs=[
            pl.BlockSpec((QN, 8), lambda i: (0, 0)),
            pl.BlockSpec((QN, 8), lambda i: (0, 0)),
        ],
        out_shape=[
            jax.ShapeDtypeStruct((QN, 8), jnp.float32),
            jax.ShapeDtypeStruct((QN, 8), jnp.int32),
        ],
        scratch_shapes=[
            pltpu.VMEM((QN, 8), jnp.float32),
            pltpu.VMEM((QN, 8), jnp.int32),
        ],
    )(q, xt)


def _sc_vote_body(labels_hbm, idx_hbm, pred_hbm, idx_v, lab_v, pred_v, sem):
    c = lax.axis_index("c")
    s = lax.axis_index("s")
    wid = s * 2 + c

    @pl.when(wid == 0)
    def _():
        pltpu.sync_copy(idx_hbm, idx_v)
        pltpu.async_copy(labels_hbm.at[idx_v], lab_v, sem).wait()
        labs = [lab_v[pl.ds(j * QN, QN)] for j in range(NNBR)]
        best = jnp.full((QN,), -1, jnp.int32)
        pred = jnp.full((QN,), 0, jnp.int32)
        one = jnp.full((QN,), 1, jnp.int32)
        zero = jnp.full((QN,), 0, jnp.int32)
        for cc in range(NCLS):
            cc_v = jnp.full((QN,), cc, jnp.int32)
            cnt = zero
            for j in range(NNBR):
                cnt = cnt + jnp.where(labs[j] == cc_v, one, zero)
            better = cnt > best
            best = jnp.where(better, cnt, best)
            pred = jnp.where(better, cc_v, pred)
        pred_v[...] = pred
        pltpu.sync_copy(pred_v, pred_hbm)


@functools.cache
def _sc_vote():
    return pl.kernel(
        _sc_vote_body,
        out_type=jax.ShapeDtypeStruct((QN,), jnp.int32),
        mesh=plsc.VectorSubcoreMesh(core_axis_name="c", subcore_axis_name="s"),
        scratch_types=[
            pltpu.VMEM((NNBR * QN,), jnp.int32),
            pltpu.VMEM((NNBR * QN,), jnp.int32),
            pltpu.VMEM((QN,), jnp.int32),
            pltpu.SemaphoreType.DMA,
        ],
    )


def kernel(test_query_embedding, train_x, train_labels):
    q = test_query_embedding
    xt = train_x.T
    outv, outi = _tc_topk(q, xt)
    neg_topk_dist = outv[:, :NNBR]
    idx_flat = outi[:, :NNBR].T.reshape(-1)

    pred = _sc_vote()(train_labels, idx_flat)
    return pred, neg_topk_dist


# speedup vs baseline: 4.3378x; 1.3431x over previous
import functools

import jax
import jax.numpy as jnp
from jax import lax
from jax.experimental import pallas as pl
from jax.experimental.pallas import tpu as pltpu
from jax.experimental.pallas import tpu_sc as plsc

QN = 16
DN = 16
KN = 1_000_000
NNBR = 5
NCLS = 32

BLK = 32768
NSPL = 8
QW = BLK // NSPL
GRID = (KN + BLK - 1) // BLK

NEGF = -3.0e38
BIGI = 2**31 - 1


def _topk_extract2(cv, ci, n):
    vs, is_ = [], []
    for _ in range(n):
        m = jnp.max(cv, axis=1, keepdims=True)
        sel = jnp.where(cv == m, ci, BIGI)
        ii = jnp.min(sel, axis=1, keepdims=True)
        vs.append(m)
        is_.append(ii)
        cv = jnp.where(ci == ii, NEGF, cv)
    return jnp.concatenate(vs, axis=1), jnp.concatenate(is_, axis=1)


def _tc_body(q_ref, xt_ref, outv_ref, outi_ref, sv_ref, si_ref):
    i = pl.program_id(0)

    @pl.when(i == 0)
    def _init():
        sv_ref[...] = jnp.full((QN, 8), NEGF, jnp.float32)
        si_ref[...] = jnp.full((QN, 8), BIGI, jnp.int32)

    xt = xt_ref[...]
    q2bf = (q_ref[...] * 2.0).astype(jnp.bfloat16)
    dots = lax.dot_general(q2bf, xt.astype(jnp.bfloat16),
                           (((1,), (0,)), ((), ())),
                           preferred_element_type=jnp.float32)
    xn = jnp.sum(xt * xt, axis=0, keepdims=True)
    neg = dots - xn

    bvs, bis = [sv_ref[:, 0:NNBR]], [si_ref[:, 0:NNBR]]
    for qd in range(NSPL):
        cv = neg[:, qd * QW:(qd + 1) * QW]
        ci = (lax.broadcasted_iota(jnp.int32, (QN, QW), 1)
              + (i * BLK + qd * QW))
        cv = jnp.where(ci < KN, cv, NEGF)
        bv, bi = _topk_extract2(cv, ci, NNBR)
        bvs.append(bv)
        bis.append(bi)
    mv = jnp.concatenate(bvs, axis=1)
    mi = jnp.concatenate(bis, axis=1)
    nv, ni = _topk_extract2(mv, mi, NNBR)
    sv_ref[:, 0:NNBR] = nv
    si_ref[:, 0:NNBR] = ni

    @pl.when(i == GRID - 1)
    def _fin():
        qq = q_ref[...]
        qn = jnp.sum(qq * qq, axis=1, keepdims=True)
        outv_ref[:, 0:NNBR] = sv_ref[:, 0:NNBR] - qn
        outi_ref[:, 0:NNBR] = si_ref[:, 0:NNBR]


def _tc_topk(q, xt):
    return pl.pallas_call(
        _tc_body,
        grid=(GRID,),
        in_specs=[
            pl.BlockSpec((QN, DN), lambda i: (0, 0)),
            pl.BlockSpec((QN, BLK), lambda i: (0, i)),
        ],
        out_specs=[
            pl.BlockSpec((QN, 8), lambda i: (0, 0)),
            pl.BlockSpec((QN, 8), lambda i: (0, 0)),
        ],
        out_shape=[
            jax.ShapeDtypeStruct((QN, 8), jnp.float32),
            jax.ShapeDtypeStruct((QN, 8), jnp.int32),
        ],
        scratch_shapes=[
            pltpu.VMEM((QN, 8), jnp.float32),
            pltpu.VMEM((QN, 8), jnp.int32),
        ],
    )(q, xt)


def _sc_vote_body(labels_hbm, idx_hbm, pred_hbm, idx_v, lab_v, pred_v, sem):
    c = lax.axis_index("c")
    s = lax.axis_index("s")
    wid = s * 2 + c

    @pl.when(wid == 0)
    def _():
        pltpu.sync_copy(idx_hbm, idx_v)
        pltpu.async_copy(labels_hbm.at[idx_v], lab_v, sem).wait()
        labs = [lab_v[pl.ds(j * QN, QN)] for j in range(NNBR)]
        best = jnp.full((QN,), -1, jnp.int32)
        pred = jnp.full((QN,), 0, jnp.int32)
        one = jnp.full((QN,), 1, jnp.int32)
        zero = jnp.full((QN,), 0, jnp.int32)
        for cc in range(NCLS):
            cc_v = jnp.full((QN,), cc, jnp.int32)
            cnt = zero
            for j in range(NNBR):
                cnt = cnt + jnp.where(labs[j] == cc_v, one, zero)
            better = cnt > best
            best = jnp.where(better, cnt, best)
            pred = jnp.where(better, cc_v, pred)
        pred_v[...] = pred
        pltpu.sync_copy(pred_v, pred_hbm)


@functools.cache
def _sc_vote():
    return pl.kernel(
        _sc_vote_body,
        out_type=jax.ShapeDtypeStruct((QN,), jnp.int32),
        mesh=plsc.VectorSubcoreMesh(core_axis_name="c", subcore_axis_name="s"),
        scratch_types=[
            pltpu.VMEM((NNBR * QN,), jnp.int32),
            pltpu.VMEM((NNBR * QN,), jnp.int32),
            pltpu.VMEM((QN,), jnp.int32),
            pltpu.SemaphoreType.DMA,
        ],
    )


def kernel(test_query_embedding, train_x, train_labels):
    q = test_query_embedding
    xt = train_x.T
    outv, outi = _tc_topk(q, xt)
    neg_topk_dist = outv[:, :NNBR]
    idx_flat = outi[:, :NNBR].T.reshape(-1)

    pred = _sc_vote()(train_labels, idx_flat)
    return pred, neg_topk_dist


# speedup vs baseline: 4.9936x; 1.1512x over previous
import functools

import jax
import jax.numpy as jnp
from jax import lax
from jax.experimental import pallas as pl
from jax.experimental.pallas import tpu as pltpu
from jax.experimental.pallas import tpu_sc as plsc

QN = 16
DN = 16
KN = 1_000_000
NNBR = 5
NCLS = 32

BLK = 65536
NSPL = 16
QW = BLK // NSPL
GRID = (KN + BLK - 1) // BLK

NEGF = -3.0e38
BIGI = 2**31 - 1


def _topk_extract2(cv, ci, n):
    vs, is_ = [], []
    for _ in range(n):
        m = jnp.max(cv, axis=1, keepdims=True)
        sel = jnp.where(cv == m, ci, BIGI)
        ii = jnp.min(sel, axis=1, keepdims=True)
        vs.append(m)
        is_.append(ii)
        cv = jnp.where(ci == ii, NEGF, cv)
    return jnp.concatenate(vs, axis=1), jnp.concatenate(is_, axis=1)


def _tc_body(q_ref, xt_ref, outv_ref, outi_ref, sv_ref, si_ref):
    i = pl.program_id(0)

    @pl.when(i == 0)
    def _init():
        sv_ref[...] = jnp.full((QN, 8), NEGF, jnp.float32)
        si_ref[...] = jnp.full((QN, 8), BIGI, jnp.int32)

    xt = xt_ref[...]
    q2bf = (q_ref[...] * 2.0).astype(jnp.bfloat16)
    dots = lax.dot_general(q2bf, xt.astype(jnp.bfloat16),
                           (((1,), (0,)), ((), ())),
                           preferred_element_type=jnp.float32)
    xn = jnp.sum(xt * xt, axis=0, keepdims=True)
    neg = dots - xn

    bvs, bis = [sv_ref[:, 0:NNBR]], [si_ref[:, 0:NNBR]]
    for qd in range(NSPL):
        cv = neg[:, qd * QW:(qd + 1) * QW]
        ci = (lax.broadcasted_iota(jnp.int32, (QN, QW), 1)
              + (i * BLK + qd * QW))
        cv = jnp.where(ci < KN, cv, NEGF)
        bv, bi = _topk_extract2(cv, ci, NNBR)
        bvs.append(bv)
        bis.append(bi)
    mv = jnp.concatenate(bvs, axis=1)
    mi = jnp.concatenate(bis, axis=1)
    nv, ni = _topk_extract2(mv, mi, NNBR)
    sv_ref[:, 0:NNBR] = nv
    si_ref[:, 0:NNBR] = ni

    @pl.when(i == GRID - 1)
    def _fin():
        qq = q_ref[...]
        qn = jnp.sum(qq * qq, axis=1, keepdims=True)
        outv_ref[:, 0:NNBR] = sv_ref[:, 0:NNBR] - qn
        outi_ref[:, 0:NNBR] = si_ref[:, 0:NNBR]


def _tc_topk(q, xt):
    return pl.pallas_call(
        _tc_body,
        grid=(GRID,),
        in_specs=[
            pl.BlockSpec((QN, DN), lambda i: (0, 0)),
            pl.BlockSpec((QN, BLK), lambda i: (0, i)),
        ],
        out_specs=[
            pl.BlockSpec((QN, 8), lambda i: (0, 0)),
            pl.BlockSpec((QN, 8), lambda i: (0, 0)),
        ],
        out_shape=[
            jax.ShapeDtypeStruct((QN, 8), jnp.float32),
            jax.ShapeDtypeStruct((QN, 8), jnp.int32),
        ],
        scratch_shapes=[
            pltpu.VMEM((QN, 8), jnp.float32),
            pltpu.VMEM((QN, 8), jnp.int32),
        ],
    )(q, xt)


def _sc_vote_body(labels_hbm, idx_hbm, pred_hbm, idx_v, lab_v, pred_v, sem):
    c = lax.axis_index("c")
    s = lax.axis_index("s")
    wid = s * 2 + c

    @pl.when(wid == 0)
    def _():
        pltpu.sync_copy(idx_hbm, idx_v)
        pltpu.async_copy(labels_hbm.at[idx_v], lab_v, sem).wait()
        labs = [lab_v[pl.ds(j * QN, QN)] for j in range(NNBR)]
        best = jnp.full((QN,), -1, jnp.int32)
        pred = jnp.full((QN,), 0, jnp.int32)
        one = jnp.full((QN,), 1, jnp.int32)
        zero = jnp.full((QN,), 0, jnp.int32)
        for cc in range(NCLS):
            cc_v = jnp.full((QN,), cc, jnp.int32)
            cnt = zero
            for j in range(NNBR):
                cnt = cnt + jnp.where(labs[j] == cc_v, one, zero)
            better = cnt > best
            best = jnp.where(better, cnt, best)
            pred = jnp.where(better, cc_v, pred)
        pred_v[...] = pred
        pltpu.sync_copy(pred_v, pred_hbm)


@functools.cache
def _sc_vote():
    return pl.kernel(
        _sc_vote_body,
        out_type=jax.ShapeDtypeStruct((QN,), jnp.int32),
        mesh=plsc.VectorSubcoreMesh(core_axis_name="c", subcore_axis_name="s"),
        scratch_types=[
            pltpu.VMEM((NNBR * QN,), jnp.int32),
            pltpu.VMEM((NNBR * QN,), jnp.int32),
            pltpu.VMEM((QN,), jnp.int32),
            pltpu.SemaphoreType.DMA,
        ],
    )


def kernel(test_query_embedding, train_x, train_labels):
    q = test_query_embedding
    xt = train_x.T
    outv, outi = _tc_topk(q, xt)
    neg_topk_dist = outv[:, :NNBR]
    idx_flat = outi[:, :NNBR].T.reshape(-1)

    pred = _sc_vote()(train_labels, idx_flat)
    return pred, neg_topk_dist


# speedup vs baseline: 5.3539x; 1.0721x over previous
import functools

import jax
import jax.numpy as jnp
from jax import lax
from jax.experimental import pallas as pl
from jax.experimental.pallas import tpu as pltpu
from jax.experimental.pallas import tpu_sc as plsc

QN = 16
DN = 16
KN = 1_000_000
NNBR = 5
NCLS = 32

BLK = 131072
NSPL = 32
QW = BLK // NSPL
GRID = (KN + BLK - 1) // BLK

NEGF = -3.0e38
BIGI = 2**31 - 1


def _topk_extract2(cv, ci, n):
    vs, is_ = [], []
    for _ in range(n):
        m = jnp.max(cv, axis=1, keepdims=True)
        sel = jnp.where(cv == m, ci, BIGI)
        ii = jnp.min(sel, axis=1, keepdims=True)
        vs.append(m)
        is_.append(ii)
        cv = jnp.where(ci == ii, NEGF, cv)
    return jnp.concatenate(vs, axis=1), jnp.concatenate(is_, axis=1)


def _tc_body(q_ref, xt_ref, outv_ref, outi_ref, sv_ref, si_ref):
    i = pl.program_id(0)

    @pl.when(i == 0)
    def _init():
        sv_ref[...] = jnp.full((QN, 8), NEGF, jnp.float32)
        si_ref[...] = jnp.full((QN, 8), BIGI, jnp.int32)

    xt = xt_ref[...]
    q2bf = (q_ref[...] * 2.0).astype(jnp.bfloat16)
    dots = lax.dot_general(q2bf, xt.astype(jnp.bfloat16),
                           (((1,), (0,)), ((), ())),
                           preferred_element_type=jnp.float32)
    xn = jnp.sum(xt * xt, axis=0, keepdims=True)
    neg = dots - xn

    bvs, bis = [sv_ref[:, 0:NNBR]], [si_ref[:, 0:NNBR]]
    for qd in range(NSPL):
        cv = neg[:, qd * QW:(qd + 1) * QW]
        ci = (lax.broadcasted_iota(jnp.int32, (QN, QW), 1)
              + (i * BLK + qd * QW))
        cv = jnp.where(ci < KN, cv, NEGF)
        bv, bi = _topk_extract2(cv, ci, NNBR)
        bvs.append(bv)
        bis.append(bi)
    mv = jnp.concatenate(bvs, axis=1)
    mi = jnp.concatenate(bis, axis=1)
    nv, ni = _topk_extract2(mv, mi, NNBR)
    sv_ref[:, 0:NNBR] = nv
    si_ref[:, 0:NNBR] = ni

    @pl.when(i == GRID - 1)
    def _fin():
        qq = q_ref[...]
        qn = jnp.sum(qq * qq, axis=1, keepdims=True)
        outv_ref[:, 0:NNBR] = sv_ref[:, 0:NNBR] - qn
        outi_ref[:, 0:NNBR] = si_ref[:, 0:NNBR]


def _tc_topk(q, xt):
    return pl.pallas_call(
        _tc_body,
        grid=(GRID,),
        in_specs=[
            pl.BlockSpec((QN, DN), lambda i: (0, 0)),
            pl.BlockSpec((QN, BLK), lambda i: (0, i)),
        ],
        out_specs=[
            pl.BlockSpec((QN, 8), lambda i: (0, 0)),
            pl.BlockSpec((QN, 8), lambda i: (0, 0)),
        ],
        out_shape=[
            jax.ShapeDtypeStruct((QN, 8), jnp.float32),
            jax.ShapeDtypeStruct((QN, 8), jnp.int32),
        ],
        scratch_shapes=[
            pltpu.VMEM((QN, 8), jnp.float32),
            pltpu.VMEM((QN, 8), jnp.int32),
        ],
    )(q, xt)


def _sc_vote_body(labels_hbm, idx_hbm, pred_hbm, idx_v, lab_v, pred_v, sem):
    c = lax.axis_index("c")
    s = lax.axis_index("s")
    wid = s * 2 + c

    @pl.when(wid == 0)
    def _():
        pltpu.sync_copy(idx_hbm, idx_v)
        pltpu.async_copy(labels_hbm.at[idx_v], lab_v, sem).wait()
        labs = [lab_v[pl.ds(j * QN, QN)] for j in range(NNBR)]
        best = jnp.full((QN,), -1, jnp.int32)
        pred = jnp.full((QN,), 0, jnp.int32)
        one = jnp.full((QN,), 1, jnp.int32)
        zero = jnp.full((QN,), 0, jnp.int32)
        for cc in range(NCLS):
            cc_v = jnp.full((QN,), cc, jnp.int32)
            cnt = zero
            for j in range(NNBR):
                cnt = cnt + jnp.where(labs[j] == cc_v, one, zero)
            better = cnt > best
            best = jnp.where(better, cnt, best)
            pred = jnp.where(better, cc_v, pred)
        pred_v[...] = pred
        pltpu.sync_copy(pred_v, pred_hbm)


@functools.cache
def _sc_vote():
    return pl.kernel(
        _sc_vote_body,
        out_type=jax.ShapeDtypeStruct((QN,), jnp.int32),
        mesh=plsc.VectorSubcoreMesh(core_axis_name="c", subcore_axis_name="s"),
        scratch_types=[
            pltpu.VMEM((NNBR * QN,), jnp.int32),
            pltpu.VMEM((NNBR * QN,), jnp.int32),
            pltpu.VMEM((QN,), jnp.int32),
            pltpu.SemaphoreType.DMA,
        ],
    )


def kernel(test_query_embedding, train_x, train_labels):
    q = test_query_embedding
    xt = train_x.T
    outv, outi = _tc_topk(q, xt)
    neg_topk_dist = outv[:, :NNBR]
    idx_flat = outi[:, :NNBR].T.reshape(-1)

    pred = _sc_vote()(train_labels, idx_flat)
    return pred, neg_topk_dist
